# baseline (speedup 1.0000x reference)
import functools

import jax
import jax.numpy as jnp
from jax import lax
from jax.experimental import pallas as pl
from jax.experimental.pallas import tpu as pltpu
from jax.experimental.pallas import tpu_sc as plsc

_B, _T, _D = 4, 4096, 1024
_NC, _NS, _L = 2, 16, 16
_NW = _NC * _NS
_ROWS_W = _T // _NW
_CT = 16
_NCH = _ROWS_W // _CT
_NBUF = 6
_LEAD = 4
_UNROLL = 8
_BLKS_ROW = _D // (_L * _UNROLL)
_STEPS = [(c, b) for c in range(_NCH) for b in range(_B)]

_mesh = plsc.VectorSubcoreMesh(core_axis_name="c", subcore_axis_name="s")


@functools.partial(
    pl.kernel,
    out_type=jax.ShapeDtypeStruct((_B, _T, _D), jnp.float32),
    mesh=_mesh,
    scratch_types=[
        [pltpu.VMEM((_CT, _D), jnp.float32) for _ in range(_NBUF)],
        [pltpu.VMEM((_CT, _D), jnp.float32) for _ in range(1)],
        [pltpu.SemaphoreType.DMA for _ in range(_NBUF)],
        [pltpu.SemaphoreType.DMA for _ in range(_NBUF)],
        [pltpu.SemaphoreType.DMA for _ in range(1)],
    ],
)
def _sc_add(x_hbm, pos_hbm, out_hbm, wb, pb, sin, sout, spos):
    wid = lax.axis_index("s") * _NC + lax.axis_index("c")
    t_base = wid * _ROWS_W

    in_cp = [None] * len(_STEPS)
    out_cp = [None] * len(_STEPS)
    pos_cp = [None] * _NCH

    def issue_pos(c):
        pos_cp[c] = pltpu.async_copy(
            pos_hbm.at[pl.ds(t_base + c * _CT, _CT)], pb[0], spos[0]
        )

    def issue_in(s):
        c, b = _STEPS[s]
        if s >= _NBUF:
            out_cp[s - _NBUF].wait()
        in_cp[s] = pltpu.async_copy(
            x_hbm.at[b, pl.ds(t_base + c * _CT, _CT)], wb[s % _NBUF], sin[s % _NBUF]
        )

    issue_pos(0)
    for s in range(_LEAD):
        issue_in(s)

    for s, (c, b) in enumerate(_STEPS):
        k = s % _NBUF
        w, p = wb[k], pb[0]
        in_cp[s].wait()
        if b == 0:
            pos_cp[c].wait()

        @plsc.parallel_loop(0, _CT * _BLKS_ROW)
        def _add(i):
            r = i // _BLKS_ROW
            j = i % _BLKS_ROW
            for u in range(_UNROLL):
                sl = pl.ds((j * _UNROLL + u) * _L, _L)
                w[r, sl] = w[r, sl] + p[r, sl]

        if b == _B - 1 and c + 1 < _NCH:
            issue_pos(c + 1)
        out_cp[s] = pltpu.async_copy(
            w, out_hbm.at[b, pl.ds(t_base + c * _CT, _CT)], sout[k]
        )
        if s + _LEAD < len(_STEPS):
            issue_in(s + _LEAD)

    for s in range(len(_STEPS) - _NBUF, len(_STEPS)):
        out_cp[s].wait()


def kernel(x, pos_table):
    return _sc_add(x, pos_table)


# speedup vs baseline: 1.1284x; 1.1284x over previous
import functools

import jax
import jax.numpy as jnp
from jax import lax
from jax.experimental import pallas as pl
from jax.experimental.pallas import tpu as pltpu
from jax.experimental.pallas import tpu_sc as plsc

_B, _T, _D = 4, 4096, 1024
_NC, _NS, _L = 2, 16, 16
_NW = _NC * _NS
_ROWS_W = _T // _NW
_CT = 16
_NCH = _ROWS_W // _CT
_NBUF = 5
_LEAD = 4
_UNROLL = 8
_BLKS_ROW = _D // (_L * _UNROLL)
_STEPS = [(c, b) for c in range(_NCH) for b in range(_B)]

_mesh = plsc.VectorSubcoreMesh(core_axis_name="c", subcore_axis_name="s")


@functools.partial(
    pl.kernel,
    out_type=jax.ShapeDtypeStruct((_B, _T, _D), jnp.float32),
    mesh=_mesh,
    scratch_types=[
        [pltpu.VMEM((_CT, _D), jnp.float32) for _ in range(_NBUF)],
        [pltpu.VMEM((_CT, _D), jnp.float32) for _ in range(2)],
        [pltpu.SemaphoreType.DMA for _ in range(_NBUF)],
        [pltpu.SemaphoreType.DMA for _ in range(_NBUF)],
        [pltpu.SemaphoreType.DMA for _ in range(2)],
    ],
)
def _sc_add(x_hbm, pos_hbm, out_hbm, wb, pb, sin, sout, spos):
    wid = lax.axis_index("s") * _NC + lax.axis_index("c")
    t_base = wid * _ROWS_W

    in_cp = [None] * len(_STEPS)
    out_cp = [None] * len(_STEPS)
    pos_cp = [None] * _NCH

    def issue_pos(c):
        pos_cp[c] = pltpu.async_copy(
            pos_hbm.at[pl.ds(t_base + c * _CT, _CT)], pb[c % 2], spos[c % 2]
        )

    def issue_in(s):
        c, b = _STEPS[s]
        if s >= _NBUF:
            out_cp[s - _NBUF].wait()
        in_cp[s] = pltpu.async_copy(
            x_hbm.at[b, pl.ds(t_base + c * _CT, _CT)], wb[s % _NBUF], sin[s % _NBUF]
        )

    issue_pos(0)
    if _NCH > 1:
        issue_pos(1)
    for s in range(_LEAD):
        issue_in(s)

    for s, (c, b) in enumerate(_STEPS):
        k = s % _NBUF
        w, p = wb[k], pb[c % 2]
        in_cp[s].wait()
        if b == 0:
            pos_cp[c].wait()

        @plsc.parallel_loop(0, _CT * _BLKS_ROW)
        def _add(i):
            r = i // _BLKS_ROW
            j = i % _BLKS_ROW
            for u in range(_UNROLL):
                sl = pl.ds((j * _UNROLL + u) * _L, _L)
                w[r, sl] = w[r, sl] + p[r, sl]

        if b == _B - 1 and c + 2 < _NCH:
            issue_pos(c + 2)
        out_cp[s] = pltpu.async_copy(
            w, out_hbm.at[b, pl.ds(t_base + c * _CT, _CT)], sout[k]
        )
        if s + _LEAD < len(_STEPS):
            issue_in(s + _LEAD)

    for s in range(len(_STEPS) - _NBUF, len(_STEPS)):
        out_cp[s].wait()


def kernel(x, pos_table):
    return _sc_add(x, pos_table)
